# baseline (speedup 1.0000x reference)
import functools

import jax
import jax.numpy as jnp
from jax import lax
from jax.experimental import pallas as pl
from jax.experimental.pallas import tpu as pltpu
from jax.experimental.pallas import tpu_sc as plsc

_N_NODES = 10000
_NC = 2
_NS = 16
_NW = _NC * _NS
_CH = 128
_KB = 1
_NRP = 10240
_ZR = _NRP // _NS


def _sc_spmm(table, src, dst, wts, zrows, d):
  ep = src.shape[0]
  per_w = ep // _NW
  n_chunks = per_w // _CH
  t_rows = table.shape[0]
  t_pad = -(-t_rows // _NS) * _NS
  if t_pad != t_rows:
    table = jnp.concatenate(
        [table, jnp.zeros((t_pad - t_rows, d), jnp.float32)])
  mesh = plsc.VectorSubcoreMesh(core_axis_name="c", subcore_axis_name="s")

  @functools.partial(
      pl.kernel,
      mesh=mesh,
      compiler_params=pltpu.CompilerParams(use_tc_tiling_on_sc=False),
      out_type=jax.ShapeDtypeStruct((_NC * _NRP, d), jnp.float32),
      scratch_types=[
          pltpu.VMEM((n_chunks, _CH), jnp.int32),
          pltpu.VMEM((n_chunks, _CH), jnp.int32),
          pltpu.VMEM((n_chunks, _CH), jnp.float32),
          pltpu.VMEM((_CH, d), jnp.float32),
          pltpu.VMEM((_CH,), jnp.int32),
          pltpu.VMEM_SHARED((_NRP, d), jnp.float32),
          pltpu.VMEM_SHARED((t_pad, d), jnp.float32),
          pltpu.SemaphoreType.DMA,
          pltpu.SemaphoreType.DMA,
      ],
  )
  def run(table_h, src_h, dst_h, w_h, zero_h, out_h,
          src_v, dst_v, w_v, rows_v, dst_flat, acc_s, tab_s, sem, sem_sc):
    c = lax.axis_index("c")
    s = lax.axis_index("s")
    wid = s * _NC + c
    pltpu.sync_copy(src_h.at[wid], src_v)
    pltpu.sync_copy(dst_h.at[wid], dst_v)
    pltpu.sync_copy(w_h.at[wid], w_v)
    pltpu.sync_copy(zero_h, acc_s.at[pl.ds(s * _ZR, _ZR)])
    tpw = t_pad // _NS
    pltpu.sync_copy(table_h.at[pl.ds(s * tpw, tpw)],
                    tab_s.at[pl.ds(s * tpw, tpw)])
    plsc.subcore_barrier()

    dnums = lax.GatherDimensionNumbers(
        offset_dims=(), collapsed_slice_dims=(0,), start_index_map=(0,))

    def step(g, carry):
      pltpu.async_copy(tab_s.at[src_v.at[g]], rows_v, sem).wait()

      def scale_grp(b, carry2):
        wvec = w_v[g, pl.ds(b * 16, 16)]
        for e in range(16):
          wbc = lax.gather(
              wvec, jnp.full((16, 1), e, jnp.int32), dnums, slice_sizes=(1,),
              mode=lax.GatherScatterMode.PROMISE_IN_BOUNDS)
          r = b * 16 + e
          for j in range(d // 16):
            rows_v[r, pl.ds(j * 16, 16)] = rows_v[r, pl.ds(j * 16, 16)] * wbc
        return carry2

      lax.fori_loop(0, _CH // 16, scale_grp, 0)
      for p in range(_CH // 16):
        dst_flat[pl.ds(p * 16, 16)] = dst_v[g, pl.ds(p * 16, 16)]
      pltpu.async_copy(rows_v, acc_s.at[dst_flat], sem_sc, add=True).wait()
      return carry

    lax.fori_loop(0, n_chunks, step, 0)

    plsc.subcore_barrier()
    pltpu.sync_copy(acc_s.at[pl.ds(s * _ZR, _ZR)],
                    out_h.at[pl.ds(c * _NRP + s * _ZR, _ZR)])

  return run(table, src.reshape(_NW, n_chunks, _CH),
             dst.reshape(_NW, n_chunks, _CH),
             wts.reshape(_NW, n_chunks, _CH), zrows)


def _sc_densify(flat_idx, vals, zrows, n_flat):
  ep = vals.shape[0]
  per_w = ep // _NW
  n_chunks = per_w // _CH
  zn = n_flat // _NS
  mesh = plsc.VectorSubcoreMesh(core_axis_name="c", subcore_axis_name="s")

  @functools.partial(
      pl.kernel,
      mesh=mesh,
      compiler_params=pltpu.CompilerParams(use_tc_tiling_on_sc=False),
      out_type=jax.ShapeDtypeStruct((_NC * n_flat,), jnp.float32),
      scratch_types=[
          pltpu.VMEM((n_chunks, _CH), jnp.int32),
          pltpu.VMEM((n_chunks, _CH), jnp.float32),
          pltpu.VMEM((_CH,), jnp.int32),
          pltpu.VMEM((_CH,), jnp.float32),
          pltpu.VMEM_SHARED((n_flat,), jnp.float32),
          pltpu.SemaphoreType.DMA,
      ],
  )
  def run(idx_h, val_h, zero_h, out_h, idx_v, val_v, idx_f, val_f, acc_s,
          sem_sc):
    c = lax.axis_index("c")
    s = lax.axis_index("s")
    wid = s * _NC + c
    pltpu.sync_copy(idx_h.at[wid], idx_v)
    pltpu.sync_copy(val_h.at[wid], val_v)
    pltpu.sync_copy(zero_h, acc_s.at[pl.ds(s * zn, zn)])
    plsc.subcore_barrier()

    def step(g, carry):
      for p in range(_CH // 16):
        idx_f[pl.ds(p * 16, 16)] = idx_v[g, pl.ds(p * 16, 16)]
        val_f[pl.ds(p * 16, 16)] = val_v[g, pl.ds(p * 16, 16)]
      pltpu.async_copy(val_f, acc_s.at[idx_f], sem_sc, add=True).wait()
      return carry

    lax.fori_loop(0, n_chunks, step, 0)
    plsc.subcore_barrier()
    pltpu.sync_copy(acc_s.at[pl.ds(s * zn, zn)],
                    out_h.at[pl.ds(c * n_flat + s * zn, zn)])

  return run(flat_idx.reshape(_NW, n_chunks, _CH),
             vals.reshape(_NW, n_chunks, _CH), zrows)


def _tc_dense_l1(p, w1, b1):
  def body(p_ref, w_ref, b_ref, o_ref):
    f = p_ref[0] + p_ref[1]
    o_ref[...] = (
        jnp.dot(f, w_ref[...], preferred_element_type=jnp.float32,
                precision=lax.Precision.HIGHEST)
        + b_ref[...]
    )
  return pl.pallas_call(
      body,
      out_shape=jax.ShapeDtypeStruct((p.shape[1], w1.shape[1]), jnp.float32),
  )(p, w1, b1)


def _tc_bias_add(p, b):
  def body(p_ref, b_ref, o_ref):
    o_ref[...] = p_ref[0] + p_ref[1] + b_ref[...]
  return pl.pallas_call(
      body,
      out_shape=jax.ShapeDtypeStruct(p.shape[1:], jnp.float32),
  )(p, b)


def _tc_relu_mm(p, w2, b2):
  def body(p_ref, w_ref, b_ref, o_ref):
    h = jnp.maximum(p_ref[0] + p_ref[1], 0.0)
    o_ref[...] = (
        jnp.dot(h, w_ref[...], preferred_element_type=jnp.float32,
                precision=lax.Precision.HIGHEST)
        + b_ref[...]
    )
  return pl.pallas_call(
      body,
      out_shape=jax.ShapeDtypeStruct((p.shape[1], w2.shape[1]), jnp.float32),
  )(p, w2, b2)


def _tc_final(p):
  def body(p_ref, o_ref):
    x = p_ref[0] + p_ref[1]
    m = jnp.max(x, axis=1, keepdims=True)
    e = jnp.exp(x - m)
    o_ref[...] = x - m - jnp.log(jnp.sum(e, axis=1, keepdims=True))
  return pl.pallas_call(
      body,
      out_shape=jax.ShapeDtypeStruct(p.shape[1:], jnp.float32),
  )(p)


def _pad_edges(idx_src, idx_dst, w):
  e = w.shape[0]
  ep = -(-e // (_NW * _CH * _KB)) * (_NW * _CH * _KB)
  pad = ep - e
  i32 = jnp.int32
  src = jnp.concatenate([idx_src.astype(i32), jnp.zeros((pad,), i32)])
  dst = jnp.concatenate([idx_dst.astype(i32), jnp.zeros((pad,), i32)])
  wp = jnp.concatenate([w, jnp.zeros((pad,), jnp.float32)])
  return src, dst, wp


def kernel(feature_indices, feature_values, edge_indices, edge_weights,
           W1, b1, W2, b2):
  hid = W1.shape[1]
  lab = W2.shape[1]
  ne = W1.shape[0]
  e_src, e_dst, e_w = _pad_edges(
      edge_indices[1], edge_indices[0], edge_weights)
  z64 = jnp.zeros((_ZR, hid), jnp.float32)
  z16 = jnp.zeros((_ZR, lab), jnp.float32)

  n_flat = _NRP * ne
  flat = (feature_indices[0].astype(jnp.int32) * ne
          + feature_indices[1].astype(jnp.int32))
  flat_p, _, f_w = _pad_edges(flat, flat, feature_values)
  zflat = jnp.zeros((n_flat // _NS,), jnp.float32)
  p1 = _sc_densify(flat_p, f_w, zflat, n_flat)
  h1 = _tc_dense_l1(p1.reshape(_NC, _NRP, ne), W1, b1.reshape(1, hid))
  p2 = _sc_spmm(h1, e_src, e_dst, e_w, z64, hid)
  h2 = _tc_relu_mm(p2.reshape(_NC, _NRP, hid), W2, b2.reshape(1, lab))
  p3 = _sc_spmm(h2, e_src, e_dst, e_w, z16, lab)
  p3 = p3.reshape(_NC, _NRP, lab)[:, :_N_NODES, :]
  return _tc_final(p3)


# speedup vs baseline: 1.0003x; 1.0003x over previous
import functools

import jax
import jax.numpy as jnp
from jax import lax
from jax.experimental import pallas as pl
from jax.experimental.pallas import tpu as pltpu
from jax.experimental.pallas import tpu_sc as plsc

_N_NODES = 10000
_NC = 2
_NS = 16
_NW = _NC * _NS
_CH = 128
_KB = 1
_NRP = 10240
_ZR = _NRP // _NS


def _sc_spmm(table, src, dst, wts, zrows, d):
  ep = src.shape[0]
  per_w = ep // _NW
  n_chunks = per_w // _CH
  t_rows = table.shape[0]
  t_pad = -(-t_rows // _NS) * _NS
  if t_pad != t_rows:
    table = jnp.concatenate(
        [table, jnp.zeros((t_pad - t_rows, d), jnp.float32)])
  mesh = plsc.VectorSubcoreMesh(core_axis_name="c", subcore_axis_name="s")

  @functools.partial(
      pl.kernel,
      mesh=mesh,
      compiler_params=pltpu.CompilerParams(use_tc_tiling_on_sc=False),
      out_type=jax.ShapeDtypeStruct((_NC * _NRP, d), jnp.float32),
      scratch_types=[
          pltpu.VMEM((n_chunks, _CH), jnp.int32),
          pltpu.VMEM((n_chunks, _CH), jnp.int32),
          pltpu.VMEM((n_chunks, _CH), jnp.float32),
          pltpu.VMEM((_CH, d), jnp.float32),
          pltpu.VMEM((_CH,), jnp.int32),
          pltpu.VMEM_SHARED((_NRP, d), jnp.float32),
          pltpu.VMEM_SHARED((t_pad, d), jnp.float32),
          pltpu.SemaphoreType.DMA,
          pltpu.SemaphoreType.DMA,
      ],
  )
  def run(table_h, src_h, dst_h, w_h, zero_h, out_h,
          src_v, dst_v, w_v, rows_v, dst_flat, acc_s, tab_s, sem, sem_sc):
    c = lax.axis_index("c")
    s = lax.axis_index("s")
    wid = s * _NC + c
    pltpu.sync_copy(src_h.at[wid], src_v)
    pltpu.sync_copy(dst_h.at[wid], dst_v)
    pltpu.sync_copy(w_h.at[wid], w_v)
    pltpu.sync_copy(zero_h, acc_s.at[pl.ds(s * _ZR, _ZR)])
    tpw = t_pad // _NS
    pltpu.sync_copy(table_h.at[pl.ds(s * tpw, tpw)],
                    tab_s.at[pl.ds(s * tpw, tpw)])
    plsc.subcore_barrier()

    dnums = lax.GatherDimensionNumbers(
        offset_dims=(), collapsed_slice_dims=(0,), start_index_map=(0,))

    def step(g, carry):
      pltpu.async_copy(tab_s.at[src_v.at[g]], rows_v, sem).wait()

      def scale_grp(b, carry2):
        wvec = w_v[g, pl.ds(b * 16, 16)]
        for e in range(16):
          wbc = lax.gather(
              wvec, jnp.full((16, 1), e, jnp.int32), dnums, slice_sizes=(1,),
              mode=lax.GatherScatterMode.PROMISE_IN_BOUNDS)
          r = b * 16 + e
          for j in range(d // 16):
            rows_v[r, pl.ds(j * 16, 16)] = rows_v[r, pl.ds(j * 16, 16)] * wbc
        return carry2

      lax.fori_loop(0, _CH // 16, scale_grp, 0)
      for p in range(_CH // 16):
        dst_flat[pl.ds(p * 16, 16)] = dst_v[g, pl.ds(p * 16, 16)]
      pltpu.async_copy(rows_v, acc_s.at[dst_flat], sem_sc, add=True).wait()
      return carry

    lax.fori_loop(0, n_chunks, step, 0)

    plsc.subcore_barrier()
    pltpu.sync_copy(acc_s.at[pl.ds(s * _ZR, _ZR)],
                    out_h.at[pl.ds(c * _NRP + s * _ZR, _ZR)])

  return run(table, src.reshape(_NW, n_chunks, _CH),
             dst.reshape(_NW, n_chunks, _CH),
             wts.reshape(_NW, n_chunks, _CH), zrows)


def _sc_densify(flat_idx, vals, zrows, n_flat):
  ep = vals.shape[0]
  per_w = ep // _NW
  n_chunks = per_w // _CH
  zn = n_flat // _NS
  mesh = plsc.VectorSubcoreMesh(core_axis_name="c", subcore_axis_name="s")

  @functools.partial(
      pl.kernel,
      mesh=mesh,
      compiler_params=pltpu.CompilerParams(use_tc_tiling_on_sc=False),
      out_type=jax.ShapeDtypeStruct((_NC * n_flat,), jnp.float32),
      scratch_types=[
          pltpu.VMEM((n_chunks, _CH), jnp.int32),
          pltpu.VMEM((n_chunks, _CH), jnp.float32),
          pltpu.VMEM((_CH,), jnp.int32),
          pltpu.VMEM((_CH,), jnp.float32),
          pltpu.VMEM_SHARED((n_flat,), jnp.float32),
          pltpu.SemaphoreType.DMA,
      ],
  )
  def run(idx_h, val_h, zero_h, out_h, idx_v, val_v, idx_f, val_f, acc_s,
          sem_sc):
    c = lax.axis_index("c")
    s = lax.axis_index("s")
    wid = s * _NC + c
    pltpu.sync_copy(idx_h.at[wid], idx_v)
    pltpu.sync_copy(val_h.at[wid], val_v)
    pltpu.sync_copy(zero_h, acc_s.at[pl.ds(s * zn, zn)])
    plsc.subcore_barrier()

    def step(g, carry):
      for p in range(_CH // 16):
        idx_f[pl.ds(p * 16, 16)] = idx_v[g, pl.ds(p * 16, 16)]
        val_f[pl.ds(p * 16, 16)] = val_v[g, pl.ds(p * 16, 16)]
      pltpu.async_copy(val_f, acc_s.at[idx_f], sem_sc, add=True).wait()
      return carry

    lax.fori_loop(0, n_chunks, step, 0)
    plsc.subcore_barrier()
    pltpu.sync_copy(acc_s.at[pl.ds(s * zn, zn)],
                    out_h.at[pl.ds(c * n_flat + s * zn, zn)])

  return run(flat_idx.reshape(_NW, n_chunks, _CH),
             vals.reshape(_NW, n_chunks, _CH), zrows)


def _tc_dense_l1(p, w1, b1):
  def body(p_ref, w_ref, b_ref, o_ref):
    f = p_ref[0] + p_ref[1]
    o_ref[...] = (
        jnp.dot(f, w_ref[...], preferred_element_type=jnp.float32,
                precision=lax.Precision.HIGHEST)
        + b_ref[...]
    )
  return pl.pallas_call(
      body,
      out_shape=jax.ShapeDtypeStruct((p.shape[1], w1.shape[1]), jnp.float32),
  )(p, w1, b1)


def _tc_relu_mm(p, w2, b2):
  def body(p_ref, w_ref, b_ref, o_ref):
    h = jnp.maximum(p_ref[0] + p_ref[1], 0.0)
    o_ref[...] = (
        jnp.dot(h, w_ref[...], preferred_element_type=jnp.float32,
                precision=lax.Precision.HIGHEST)
        + b_ref[...]
    )
  return pl.pallas_call(
      body,
      out_shape=jax.ShapeDtypeStruct((p.shape[1], w2.shape[1]), jnp.float32),
  )(p, w2, b2)


def _tc_final(p):
  def body(p_ref, o_ref):
    x = p_ref[0] + p_ref[1]
    m = jnp.max(x, axis=1, keepdims=True)
    e = jnp.exp(x - m)
    o_ref[...] = x - m - jnp.log(jnp.sum(e, axis=1, keepdims=True))
  return pl.pallas_call(
      body,
      out_shape=jax.ShapeDtypeStruct(p.shape[1:], jnp.float32),
  )(p)


def _pad_edges(idx_src, idx_dst, w):
  e = w.shape[0]
  ep = -(-e // (_NW * _CH * _KB)) * (_NW * _CH * _KB)
  pad = ep - e
  i32 = jnp.int32
  src = jnp.concatenate([idx_src.astype(i32), jnp.zeros((pad,), i32)])
  dst = jnp.concatenate([idx_dst.astype(i32), jnp.zeros((pad,), i32)])
  wp = jnp.concatenate([w, jnp.zeros((pad,), jnp.float32)])
  return src, dst, wp


def kernel(feature_indices, feature_values, edge_indices, edge_weights,
           W1, b1, W2, b2):
  hid = W1.shape[1]
  lab = W2.shape[1]
  ne = W1.shape[0]
  e_src, e_dst, e_w = _pad_edges(
      edge_indices[1], edge_indices[0], edge_weights)
  z64 = jnp.zeros((_ZR, hid), jnp.float32)
  z16 = jnp.zeros((_ZR, lab), jnp.float32)

  n_flat = _NRP * ne
  flat = (feature_indices[0].astype(jnp.int32) * ne
          + feature_indices[1].astype(jnp.int32))
  flat_p, _, f_w = _pad_edges(flat, flat, feature_values)
  zflat = jnp.zeros((n_flat // _NS,), jnp.float32)
  p1 = _sc_densify(flat_p, f_w, zflat, n_flat)
  h1 = _tc_dense_l1(p1.reshape(_NC, _NRP, ne), W1, b1.reshape(1, hid))
  p2 = _sc_spmm(h1, e_src, e_dst, e_w, z64, hid)
  h2 = _tc_relu_mm(p2.reshape(_NC, _NRP, hid), W2, b2.reshape(1, lab))
  p3 = _sc_spmm(h2, e_src, e_dst, e_w, z16, lab)
  p3 = p3.reshape(_NC, _NRP, lab)[:, :_N_NODES, :]
  return _tc_final(p3)
